# baseline (speedup 1.0000x reference)
import functools

import jax
import jax.numpy as jnp
from jax import lax
from jax.experimental import pallas as pl
from jax.experimental.pallas import tpu as pltpu
from jax.experimental.pallas import tpu_sc as plsc

NC = 2
NS = 16
LANES = 16
CH = 128
SUP = 4


def _mesh():
    return plsc.VectorSubcoreMesh(
        core_axis_name="c", subcore_axis_name="s", num_cores=NC, num_subcores=NS
    )


def _p1_linear(x, W, b):
    N, C = x.shape
    BLK = 2000

    def body(x_ref, w_ref, b_ref, h_ref):
        h_ref[...] = jnp.dot(x_ref[...], w_ref[...],
                             preferred_element_type=jnp.float32) + b_ref[...]

    return pl.pallas_call(
        body,
        grid=(N // BLK,),
        in_specs=[
            pl.BlockSpec((BLK, C), lambda i: (i, 0)),
            pl.BlockSpec((C, C), lambda i: (0, 0)),
            pl.BlockSpec((1, C), lambda i: (0, 0)),
        ],
        out_specs=pl.BlockSpec((BLK, C), lambda i: (i, 0)),
        out_shape=jax.ShapeDtypeStruct((N, C), jnp.float32),
    )(x, W, b)


def _padd(p0, p1, n_out, out_dtype=jnp.float32):
    rows, C = p0.shape
    BLK = next(b for b in (2528, 2000, 1264, 1000, 632, 200, 8)
               if n_out % b == 0)

    def body(a_ref, b_ref, o_ref):
        o_ref[...] = (a_ref[...] + b_ref[...]).astype(out_dtype)

    return pl.pallas_call(
        body,
        grid=(n_out // BLK,),
        in_specs=[
            pl.BlockSpec((BLK, C), lambda i: (i, 0)),
            pl.BlockSpec((BLK, C), lambda i: (i, 0)),
        ],
        out_specs=pl.BlockSpec((BLK, C), lambda i: (i, 0)),
        out_shape=jax.ShapeDtypeStruct((n_out, C), out_dtype),
    )(p0, p1)


def _p2_scatter(h, src3, dst3, zrows, n_sup, n_acc, rows_t):
    N, C = h.shape

    @functools.partial(
        pl.kernel,
        out_type=[
            jax.ShapeDtypeStruct((n_acc, C), jnp.float32),
            jax.ShapeDtypeStruct((n_acc, C), jnp.float32),
        ],
        mesh=_mesh(),
        scratch_types=[
            pltpu.VMEM((SUP, CH), jnp.int32),
            pltpu.VMEM((SUP, CH), jnp.int32),
            pltpu.VMEM((CH, C), jnp.float32),
            pltpu.VMEM((CH, C), jnp.float32),
            pltpu.VMEM_SHARED((n_acc, C), jnp.float32),
            pltpu.SemaphoreType.DMA,
            pltpu.SemaphoreType.DMA,
            pltpu.SemaphoreType.DMA,
            pltpu.SemaphoreType.DMA,
        ],
    )
    def k(h_hbm, src_hbm, dst_hbm, z_hbm, p0_hbm, p1_hbm,
          sidx, didx, rows0, rows1, acc, semg0, semg1, semw0, semw1):
        rowsv = (rows0, rows1)
        semgv = (semg0, semg1)
        semwv = (semw0, semw1)
        c = lax.axis_index("c")
        s = lax.axis_index("s")
        dummy = h_hbm.at[pl.ds(0, CH)]
        pltpu.sync_copy(z_hbm, acc.at[pl.ds(s * rows_t, rows_t)])
        plsc.subcore_barrier()

        def sup_body(g, _):
            gsc = (c * NS + s) * n_sup + g
            pltpu.sync_copy(src_hbm.at[gsc], sidx)
            pltpu.sync_copy(dst_hbm.at[gsc], didx)

            @pl.when(g > 0)
            def _():
                for b in range(2):
                    pltpu.make_async_copy(dummy, rowsv[b], semwv[b]).wait()

            for b in range(2):
                pltpu.async_copy(h_hbm.at[sidx.at[b]], rowsv[b], semgv[b])
            for j in range(SUP):
                b = j % 2
                if j >= 2:
                    pltpu.make_async_copy(dummy, rowsv[b], semwv[b]).wait()
                    pltpu.async_copy(h_hbm.at[sidx.at[j]], rowsv[b],
                                     semgv[b])
                pltpu.make_async_copy(dummy, rowsv[b], semgv[b]).wait()
                pltpu.async_copy(rowsv[b], acc.at[didx.at[j]], semwv[b],
                                 add=True)
            return 0

        lax.fori_loop(0, n_sup, sup_body, 0)
        for b in range(2):
            pltpu.make_async_copy(dummy, rowsv[b], semwv[b]).wait()
        plsc.subcore_barrier()

        @pl.when(c == 0)
        def _():
            pltpu.sync_copy(acc.at[pl.ds(s * rows_t, rows_t)],
                            p0_hbm.at[pl.ds(s * rows_t, rows_t)])

        @pl.when(c == 1)
        def _():
            pltpu.sync_copy(acc.at[pl.ds(s * rows_t, rows_t)],
                            p1_hbm.at[pl.ds(s * rows_t, rows_t)])

    return k(h, src3, dst3, zrows)


def _p3_gather(emb, src3, dst3, n_sup, e_pad, rows_t):
    N, C = emb.shape
    dt = emb.dtype

    @functools.partial(
        pl.kernel,
        out_type=[
            jax.ShapeDtypeStruct((e_pad, C), dt),
            jax.ShapeDtypeStruct((e_pad, C), dt),
        ],
        mesh=_mesh(),
        scratch_types=[
            pltpu.VMEM((SUP, CH), jnp.int32),
            pltpu.VMEM((SUP, CH), jnp.int32),
            pltpu.VMEM((CH, C), dt),
            pltpu.VMEM((CH, C), dt),
            pltpu.VMEM_SHARED((NS * rows_t, C), jnp.float32),
            pltpu.SemaphoreType.DMA,
            pltpu.SemaphoreType.DMA,
        ],
    )
    def k(e_hbm, src_hbm, dst_hbm, f1_hbm, f2_hbm,
          sidx, didx, rows1, rows2, etab, semg, semg2):
        c = lax.axis_index("c")
        s = lax.axis_index("s")
        pltpu.sync_copy(e_hbm.at[pl.ds(s * rows_t, rows_t)],
                        etab.at[pl.ds(s * rows_t, rows_t)])
        plsc.subcore_barrier()

        def sup_body(g, _):
            gsc = (c * NS + s) * n_sup + g
            pltpu.sync_copy(src_hbm.at[gsc], sidx)
            pltpu.sync_copy(dst_hbm.at[gsc], didx)

            def chunk(j, _):
                off = (gsc * SUP + j) * CH
                g1 = pltpu.async_copy(etab.at[sidx.at[j]], rows1, semg)
                g2 = pltpu.async_copy(etab.at[didx.at[j]], rows2, semg2)
                g1.wait()
                pltpu.sync_copy(rows1, f1_hbm.at[pl.ds(off, CH)])
                g2.wait()
                pltpu.sync_copy(rows2, f2_hbm.at[pl.ds(off, CH)])
                return 0

            lax.fori_loop(0, SUP, chunk, 0)
            return 0

        lax.fori_loop(0, n_sup, sup_body, 0)

    return k(emb, src3, dst3)


def _p4_mlp(f1, f2, Wp, b1r, w2r, b2r):
    e_pad, C = f1.shape
    H = Wp.shape[2]
    BLK = 4096

    def body(f1_ref, f2_ref, w_ref, b1_ref, w2_ref, b2_ref, out_ref):
        f1b = f1_ref[...].astype(jnp.bfloat16)
        f2b = f2_ref[...].astype(jnp.bfloat16)
        dn = (((0,), (1,)), ((), ()))
        acc = lax.dot_general(w_ref[0], f1b, dn,
                              preferred_element_type=jnp.float32)
        acc += lax.dot_general(w_ref[1], f2b, dn,
                               preferred_element_type=jnp.float32)
        hbT = jnp.maximum(acc + b1_ref[...], 0.0)
        ewT = lax.dot_general(w2_ref[...], hbT, (((1,), (0,)), ((), ())),
                              preferred_element_type=jnp.float32)
        ew = jnp.maximum(ewT + b2_ref[0], 0.0)
        out_ref[...] = jnp.reshape(ew, (1, 1, BLK))

    return pl.pallas_call(
        body,
        grid=(e_pad // BLK,),
        in_specs=[
            pl.BlockSpec((BLK, C), lambda i: (i, 0)),
            pl.BlockSpec((BLK, C), lambda i: (i, 0)),
            pl.BlockSpec((2, C, H), lambda i: (0, 0, 0)),
            pl.BlockSpec((H, 1), lambda i: (0, 0)),
            pl.BlockSpec((1, H), lambda i: (0, 0)),
            pl.BlockSpec(memory_space=pltpu.MemorySpace.SMEM),
        ],
        out_specs=pl.BlockSpec((1, 1, BLK), lambda i: (i, 0, 0)),
        out_shape=jax.ShapeDtypeStruct((e_pad // BLK, 1, BLK), jnp.float32),
    )(f1, f2, Wp, b1r, w2r, b2r)


def _p5_weighted_scatter(h, src3, dst3, ew3, zrows, n_sup, n_acc, rows_t):
    N, C = h.shape

    @functools.partial(
        pl.kernel,
        out_type=[
            jax.ShapeDtypeStruct((n_acc, C), jnp.float32),
            jax.ShapeDtypeStruct((n_acc, C), jnp.float32),
        ],
        mesh=_mesh(),
        scratch_types=[
            pltpu.VMEM((SUP, CH), jnp.int32),
            pltpu.VMEM((SUP, CH), jnp.int32),
            pltpu.VMEM((SUP, CH), jnp.float32),
            pltpu.VMEM((CH, C), jnp.float32),
            pltpu.VMEM((CH, C), jnp.float32),
            pltpu.VMEM_SHARED((n_acc, C), jnp.float32),
            pltpu.SemaphoreType.DMA,
            pltpu.SemaphoreType.DMA,
            pltpu.SemaphoreType.DMA,
            pltpu.SemaphoreType.DMA,
        ],
        compiler_params=pltpu.CompilerParams(needs_layout_passes=False),
    )
    def k(h_hbm, src_hbm, dst_hbm, ew_hbm, z_hbm, p0_hbm, p1_hbm,
          sidx, didx, ewb, rows0, rows1, acc,
          semg0, semg1, semw0, semw1):
        rowsv = (rows0, rows1)
        semgv = (semg0, semg1)
        semwv = (semw0, semw1)
        c = lax.axis_index("c")
        s = lax.axis_index("s")
        dummy = h_hbm.at[pl.ds(0, CH)]
        pltpu.sync_copy(z_hbm, acc.at[pl.ds(s * rows_t, rows_t)])
        plsc.subcore_barrier()

        def sup_body(g, _):
            gsc = (c * NS + s) * n_sup + g
            pltpu.sync_copy(src_hbm.at[gsc], sidx)
            pltpu.sync_copy(dst_hbm.at[gsc], didx)
            pltpu.sync_copy(ew_hbm.at[gsc], ewb)

            @pl.when(g > 0)
            def _():
                for b in range(2):
                    pltpu.make_async_copy(dummy, rowsv[b], semwv[b]).wait()

            for b in range(2):
                pltpu.async_copy(h_hbm.at[sidx.at[b]], rowsv[b], semgv[b])
            for j in range(SUP):
                b = j % 2
                rows = rowsv[b]
                if j >= 2:
                    pltpu.make_async_copy(dummy, rows, semwv[b]).wait()
                    pltpu.async_copy(h_hbm.at[sidx.at[j]], rows, semgv[b])
                pltpu.make_async_copy(dummy, rows, semgv[b]).wait()
                jv = jnp.full((LANES,), j, jnp.int32)

                @plsc.parallel_loop(0, CH, unroll=4)
                def row(r):
                    sv = plsc.load_gather(
                        ewb, [jv, jnp.full((LANES,), r, jnp.int32)])
                    for q in range(C // LANES):
                        sl = pl.ds(q * LANES, LANES)
                        rows[r, sl] = rows[r, sl] * sv

                pltpu.async_copy(rows, acc.at[didx.at[j]], semwv[b],
                                 add=True)
            return 0

        lax.fori_loop(0, n_sup, sup_body, 0)
        for b in range(2):
            pltpu.make_async_copy(dummy, rowsv[b], semwv[b]).wait()
        plsc.subcore_barrier()

        @pl.when(c == 0)
        def _():
            pltpu.sync_copy(acc.at[pl.ds(s * rows_t, rows_t)],
                            p0_hbm.at[pl.ds(s * rows_t, rows_t)])

        @pl.when(c == 1)
        def _():
            pltpu.sync_copy(acc.at[pl.ds(s * rows_t, rows_t)],
                            p1_hbm.at[pl.ds(s * rows_t, rows_t)])

    return k(h, src3, dst3, ew3, zrows)


def kernel(x, edge_index, W_base, b_base, W1, b1, W2, b2):
    N, C = x.shape
    E = edge_index.shape[1]
    H = W1.shape[1]

    n_sup = -(-E // (NC * NS * CH * SUP))
    n_chunks = n_sup * SUP
    e_pad = NC * NS * n_chunks * CH
    rows_t = 8 * (-(-(N + 1) // (NS * 8)))
    n_acc = NS * rows_t

    src = edge_index[0].astype(jnp.int32)
    dst = edge_index[1].astype(jnp.int32)
    pad = e_pad - E
    srcp = jnp.concatenate([src, jnp.zeros((pad,), jnp.int32)])
    dstp = jnp.concatenate([dst, jnp.full((pad,), N, jnp.int32)])
    src3 = srcp.reshape(-1, SUP, CH)
    dst3 = dstp.reshape(-1, SUP, CH)
    zrows = jnp.zeros((rows_t, C), jnp.float32)

    h = _p1_linear(x, W_base, b_base.reshape(1, C))
    ep0, ep1 = _p2_scatter(h, src3, dst3, zrows, n_sup, n_acc, rows_t)
    emb = _padd(ep0, ep1, n_acc)
    f1, f2 = _p3_gather(emb, src3, dst3, n_sup, e_pad, rows_t)

    Wp = jnp.stack([W1[:C], W1[C:]]).astype(jnp.bfloat16)
    ew_mat = _p4_mlp(f1, f2, Wp, b1.reshape(H, 1), W2.reshape(1, H),
                     b2.reshape(1))
    ew3 = ew_mat.reshape(-1, SUP, CH)

    lp0, lp1 = _p5_weighted_scatter(h, src3, dst3, ew3, zrows,
                                    n_sup, n_acc, rows_t)
    return _padd(lp0, lp1, N)


# speedup vs baseline: 1.0867x; 1.0867x over previous
import functools

import jax
import jax.numpy as jnp
from jax import lax
from jax.experimental import pallas as pl
from jax.experimental.pallas import tpu as pltpu
from jax.experimental.pallas import tpu_sc as plsc

NC = 2
NS = 16
LANES = 16
CH = 128
SUP = 4


def _mesh():
    return plsc.VectorSubcoreMesh(
        core_axis_name="c", subcore_axis_name="s", num_cores=NC, num_subcores=NS
    )


def _p1_linear(x, W, b):
    N, C = x.shape
    BLK = 2000

    def body(x_ref, w_ref, b_ref, h_ref):
        h_ref[...] = jnp.dot(x_ref[...], w_ref[...],
                             preferred_element_type=jnp.float32) + b_ref[...]

    return pl.pallas_call(
        body,
        grid=(N // BLK,),
        in_specs=[
            pl.BlockSpec((BLK, C), lambda i: (i, 0)),
            pl.BlockSpec((C, C), lambda i: (0, 0)),
            pl.BlockSpec((1, C), lambda i: (0, 0)),
        ],
        out_specs=pl.BlockSpec((BLK, C), lambda i: (i, 0)),
        out_shape=jax.ShapeDtypeStruct((N, C), jnp.float32),
    )(x, W, b)


def _padd(p0, p1, n_out, out_dtype=jnp.float32):
    rows, C = p0.shape
    BLK = next(b for b in (2528, 2000, 1264, 1000, 632, 200, 8)
               if n_out % b == 0)

    def body(a_ref, b_ref, o_ref):
        o_ref[...] = (a_ref[...] + b_ref[...]).astype(out_dtype)

    return pl.pallas_call(
        body,
        grid=(n_out // BLK,),
        in_specs=[
            pl.BlockSpec((BLK, C), lambda i: (i, 0)),
            pl.BlockSpec((BLK, C), lambda i: (i, 0)),
        ],
        out_specs=pl.BlockSpec((BLK, C), lambda i: (i, 0)),
        out_shape=jax.ShapeDtypeStruct((n_out, C), out_dtype),
    )(p0, p1)


def _p2_scatter(h, src3, dst3, zrows, n_sup, n_acc, rows_t, k0, k1):
    N, C = h.shape

    @functools.partial(
        pl.kernel,
        out_type=[
            jax.ShapeDtypeStruct((n_acc, C), jnp.float32),
            jax.ShapeDtypeStruct((n_acc, C), jnp.float32),
        ],
        mesh=_mesh(),
        scratch_types=[
            pltpu.VMEM((SUP, CH), jnp.int32),
            pltpu.VMEM((SUP, CH), jnp.int32),
            pltpu.VMEM((CH, C), jnp.float32),
            pltpu.VMEM((CH, C), jnp.float32),
            pltpu.VMEM_SHARED((n_acc, C), jnp.float32),
            pltpu.SemaphoreType.DMA,
            pltpu.SemaphoreType.DMA,
            pltpu.SemaphoreType.DMA,
            pltpu.SemaphoreType.DMA,
        ],
    )
    def k(h_hbm, src_hbm, dst_hbm, z_hbm, p0_hbm, p1_hbm,
          sidx, didx, rows0, rows1, acc, semg0, semg1, semw0, semw1):
        rowsv = (rows0, rows1)
        semgv = (semg0, semg1)
        semwv = (semw0, semw1)
        c = lax.axis_index("c")
        s = lax.axis_index("s")
        dummy = h_hbm.at[pl.ds(0, CH)]
        pltpu.sync_copy(z_hbm, acc.at[pl.ds(s * rows_t, rows_t)])
        plsc.subcore_barrier()
        n_mine = jnp.where(c == 0, k0, k1)
        start = jnp.where(c == 0, s * k0, NS * k0 + s * k1)

        def sup_body(g, _):
            gsc = start + g
            pltpu.sync_copy(src_hbm.at[gsc], sidx)
            pltpu.sync_copy(dst_hbm.at[gsc], didx)

            @pl.when(g > 0)
            def _():
                for b in range(2):
                    pltpu.make_async_copy(dummy, rowsv[b], semwv[b]).wait()

            for b in range(2):
                pltpu.async_copy(h_hbm.at[sidx.at[b]], rowsv[b], semgv[b])
            for j in range(SUP):
                b = j % 2
                if j >= 2:
                    pltpu.make_async_copy(dummy, rowsv[b], semwv[b]).wait()
                    pltpu.async_copy(h_hbm.at[sidx.at[j]], rowsv[b],
                                     semgv[b])
                pltpu.make_async_copy(dummy, rowsv[b], semgv[b]).wait()
                pltpu.async_copy(rowsv[b], acc.at[didx.at[j]], semwv[b],
                                 add=True)
            return 0

        lax.fori_loop(0, n_mine, sup_body, 0)

        @pl.when(n_mine > 0)
        def _():
            for b in range(2):
                pltpu.make_async_copy(dummy, rowsv[b], semwv[b]).wait()
        plsc.subcore_barrier()

        @pl.when(c == 0)
        def _():
            pltpu.sync_copy(acc.at[pl.ds(s * rows_t, rows_t)],
                            p0_hbm.at[pl.ds(s * rows_t, rows_t)])

        @pl.when(c == 1)
        def _():
            pltpu.sync_copy(acc.at[pl.ds(s * rows_t, rows_t)],
                            p1_hbm.at[pl.ds(s * rows_t, rows_t)])

    return k(h, src3, dst3, zrows)


def _p3_gather(emb, src3, dst3, n_sup, e_pad, rows_t):
    N, C = emb.shape
    dt = emb.dtype

    @functools.partial(
        pl.kernel,
        out_type=[
            jax.ShapeDtypeStruct((e_pad, C), dt),
            jax.ShapeDtypeStruct((e_pad, C), dt),
        ],
        mesh=_mesh(),
        scratch_types=[
            pltpu.VMEM((SUP, CH), jnp.int32),
            pltpu.VMEM((SUP, CH), jnp.int32),
            pltpu.VMEM((CH, C), dt),
            pltpu.VMEM((CH, C), dt),
            pltpu.VMEM_SHARED((NS * rows_t, C), jnp.float32),
            pltpu.SemaphoreType.DMA,
            pltpu.SemaphoreType.DMA,
        ],
    )
    def k(e_hbm, src_hbm, dst_hbm, f1_hbm, f2_hbm,
          sidx, didx, rows1, rows2, etab, semg, semg2):
        c = lax.axis_index("c")
        s = lax.axis_index("s")
        pltpu.sync_copy(e_hbm.at[pl.ds(s * rows_t, rows_t)],
                        etab.at[pl.ds(s * rows_t, rows_t)])
        plsc.subcore_barrier()

        def sup_body(g, _):
            gsc = (c * NS + s) * n_sup + g
            pltpu.sync_copy(src_hbm.at[gsc], sidx)
            pltpu.sync_copy(dst_hbm.at[gsc], didx)

            def chunk(j, _):
                off = (gsc * SUP + j) * CH
                g1 = pltpu.async_copy(etab.at[sidx.at[j]], rows1, semg)
                g2 = pltpu.async_copy(etab.at[didx.at[j]], rows2, semg2)
                g1.wait()
                pltpu.sync_copy(rows1, f1_hbm.at[pl.ds(off, CH)])
                g2.wait()
                pltpu.sync_copy(rows2, f2_hbm.at[pl.ds(off, CH)])
                return 0

            lax.fori_loop(0, SUP, chunk, 0)
            return 0

        lax.fori_loop(0, n_sup, sup_body, 0)

    return k(emb, src3, dst3)


def _p4_mlp(f1, f2, Wp, b1r, w2r, b2r):
    e_pad, C = f1.shape
    H = Wp.shape[2]
    BLK = 4096

    def body(f1_ref, f2_ref, w_ref, b1_ref, w2_ref, b2_ref, out_ref):
        f1b = f1_ref[...].astype(jnp.bfloat16)
        f2b = f2_ref[...].astype(jnp.bfloat16)
        dn = (((0,), (1,)), ((), ()))
        acc = lax.dot_general(w_ref[0], f1b, dn,
                              preferred_element_type=jnp.float32)
        acc += lax.dot_general(w_ref[1], f2b, dn,
                               preferred_element_type=jnp.float32)
        hbT = jnp.maximum(acc + b1_ref[...], 0.0)
        ewT = lax.dot_general(w2_ref[...], hbT, (((1,), (0,)), ((), ())),
                              preferred_element_type=jnp.float32)
        ew = jnp.maximum(ewT + b2_ref[0], 0.0)
        out_ref[...] = jnp.reshape(ew, (1, 1, BLK))

    return pl.pallas_call(
        body,
        grid=(e_pad // BLK,),
        in_specs=[
            pl.BlockSpec((BLK, C), lambda i: (i, 0)),
            pl.BlockSpec((BLK, C), lambda i: (i, 0)),
            pl.BlockSpec((2, C, H), lambda i: (0, 0, 0)),
            pl.BlockSpec((H, 1), lambda i: (0, 0)),
            pl.BlockSpec((1, H), lambda i: (0, 0)),
            pl.BlockSpec(memory_space=pltpu.MemorySpace.SMEM),
        ],
        out_specs=pl.BlockSpec((1, 1, BLK), lambda i: (i, 0, 0)),
        out_shape=jax.ShapeDtypeStruct((e_pad // BLK, 1, BLK), jnp.float32),
    )(f1, f2, Wp, b1r, w2r, b2r)


def _p5_weighted_scatter(h, src3, dst3, ew3, zrows, n_sup, n_acc, rows_t,
                         k0, k1):
    N, C = h.shape

    @functools.partial(
        pl.kernel,
        out_type=[
            jax.ShapeDtypeStruct((n_acc, C), jnp.float32),
            jax.ShapeDtypeStruct((n_acc, C), jnp.float32),
        ],
        mesh=_mesh(),
        scratch_types=[
            pltpu.VMEM((SUP, CH), jnp.int32),
            pltpu.VMEM((SUP, CH), jnp.int32),
            pltpu.VMEM((SUP, CH), jnp.float32),
            pltpu.VMEM((CH, C), jnp.float32),
            pltpu.VMEM((CH, C), jnp.float32),
            pltpu.VMEM_SHARED((n_acc, C), jnp.float32),
            pltpu.SemaphoreType.DMA,
            pltpu.SemaphoreType.DMA,
            pltpu.SemaphoreType.DMA,
            pltpu.SemaphoreType.DMA,
        ],
        compiler_params=pltpu.CompilerParams(needs_layout_passes=False),
    )
    def k(h_hbm, src_hbm, dst_hbm, ew_hbm, z_hbm, p0_hbm, p1_hbm,
          sidx, didx, ewb, rows0, rows1, acc,
          semg0, semg1, semw0, semw1):
        rowsv = (rows0, rows1)
        semgv = (semg0, semg1)
        semwv = (semw0, semw1)
        c = lax.axis_index("c")
        s = lax.axis_index("s")
        dummy = h_hbm.at[pl.ds(0, CH)]
        pltpu.sync_copy(z_hbm, acc.at[pl.ds(s * rows_t, rows_t)])
        plsc.subcore_barrier()
        n_mine = jnp.where(c == 0, k0, k1)
        start = jnp.where(c == 0, s * k0, NS * k0 + s * k1)

        def sup_body(g, _):
            gsc = start + g
            pltpu.sync_copy(src_hbm.at[gsc], sidx)
            pltpu.sync_copy(dst_hbm.at[gsc], didx)
            pltpu.sync_copy(ew_hbm.at[gsc], ewb)

            @pl.when(g > 0)
            def _():
                for b in range(2):
                    pltpu.make_async_copy(dummy, rowsv[b], semwv[b]).wait()

            for b in range(2):
                pltpu.async_copy(h_hbm.at[sidx.at[b]], rowsv[b], semgv[b])
            for j in range(SUP):
                b = j % 2
                rows = rowsv[b]
                if j >= 2:
                    pltpu.make_async_copy(dummy, rows, semwv[b]).wait()
                    pltpu.async_copy(h_hbm.at[sidx.at[j]], rows, semgv[b])
                pltpu.make_async_copy(dummy, rows, semgv[b]).wait()
                jv = jnp.full((LANES,), j, jnp.int32)

                @plsc.parallel_loop(0, CH, unroll=4)
                def row(r):
                    sv = plsc.load_gather(
                        ewb, [jv, jnp.full((LANES,), r, jnp.int32)])
                    for q in range(C // LANES):
                        sl = pl.ds(q * LANES, LANES)
                        rows[r, sl] = rows[r, sl] * sv

                pltpu.async_copy(rows, acc.at[didx.at[j]], semwv[b],
                                 add=True)
            return 0

        lax.fori_loop(0, n_mine, sup_body, 0)
        for b in range(2):
            pltpu.make_async_copy(dummy, rowsv[b], semwv[b]).wait()
        plsc.subcore_barrier()

        @pl.when(c == 0)
        def _():
            pltpu.sync_copy(acc.at[pl.ds(s * rows_t, rows_t)],
                            p0_hbm.at[pl.ds(s * rows_t, rows_t)])

        @pl.when(c == 1)
        def _():
            pltpu.sync_copy(acc.at[pl.ds(s * rows_t, rows_t)],
                            p1_hbm.at[pl.ds(s * rows_t, rows_t)])

    return k(h, src3, dst3, ew3, zrows)


def kernel(x, edge_index, W_base, b_base, W1, b1, W2, b2):
    N, C = x.shape
    E = edge_index.shape[1]
    H = W1.shape[1]

    n_sup = -(-E // (NC * NS * CH * SUP))
    n_chunks = n_sup * SUP
    e_pad = NC * NS * n_chunks * CH
    rows_t = 8 * (-(-(N + 1) // (NS * 8)))
    n_acc = NS * rows_t

    src = edge_index[0].astype(jnp.int32)
    dst = edge_index[1].astype(jnp.int32)
    pad = e_pad - E
    srcp = jnp.concatenate([src, jnp.zeros((pad,), jnp.int32)])
    dstp = jnp.concatenate([dst, jnp.full((pad,), N, jnp.int32)])
    src3 = srcp.reshape(-1, SUP, CH)
    dst3 = dstp.reshape(-1, SUP, CH)
    zrows = jnp.zeros((rows_t, C), jnp.float32)

    total_sup = NC * n_sup
    k0 = (7 * total_sup) // 10
    k1 = total_sup - k0

    h = _p1_linear(x, W_base, b_base.reshape(1, C))
    ep0, ep1 = _p2_scatter(h, src3, dst3, zrows, n_sup, n_acc, rows_t,
                           k0, k1)
    emb = _padd(ep0, ep1, n_acc)
    f1, f2 = _p3_gather(emb, src3, dst3, n_sup, e_pad, rows_t)

    Wp = jnp.stack([W1[:C], W1[C:]]).astype(jnp.bfloat16)
    ew_mat = _p4_mlp(f1, f2, Wp, b1.reshape(H, 1), W2.reshape(1, H),
                     b2.reshape(1))
    ew3 = ew_mat.reshape(-1, SUP, CH)

    lp0, lp1 = _p5_weighted_scatter(h, src3, dst3, ew3, zrows,
                                    n_sup, n_acc, rows_t, k0, k1)
    return _padd(lp0, lp1, N)


# speedup vs baseline: 1.2355x; 1.1369x over previous
import functools

import jax
import jax.numpy as jnp
from jax import lax
from jax.experimental import pallas as pl
from jax.experimental.pallas import tpu as pltpu
from jax.experimental.pallas import tpu_sc as plsc

NC = 2
NS = 16
LANES = 16
CH = 128
SUP = 4


def _mesh():
    return plsc.VectorSubcoreMesh(
        core_axis_name="c", subcore_axis_name="s", num_cores=NC, num_subcores=NS
    )


def _p1_linear(x, W, b):
    N, C = x.shape
    BLK = 2000

    def body(x_ref, w_ref, b_ref, h_ref):
        h_ref[...] = jnp.dot(x_ref[...], w_ref[...],
                             preferred_element_type=jnp.float32) + b_ref[...]

    return pl.pallas_call(
        body,
        grid=(N // BLK,),
        in_specs=[
            pl.BlockSpec((BLK, C), lambda i: (i, 0)),
            pl.BlockSpec((C, C), lambda i: (0, 0)),
            pl.BlockSpec((1, C), lambda i: (0, 0)),
        ],
        out_specs=pl.BlockSpec((BLK, C), lambda i: (i, 0)),
        out_shape=jax.ShapeDtypeStruct((N, C), jnp.float32),
    )(x, W, b)


def _padd(p0, p1, n_out, out_dtype=jnp.float32):
    rows, C = p0.shape
    BLK = next(b for b in (2528, 2000, 1264, 1000, 632, 200, 8)
               if n_out % b == 0)

    def body(a_ref, b_ref, o_ref):
        o_ref[...] = (a_ref[...] + b_ref[...]).astype(out_dtype)

    return pl.pallas_call(
        body,
        grid=(n_out // BLK,),
        in_specs=[
            pl.BlockSpec((BLK, C), lambda i: (i, 0)),
            pl.BlockSpec((BLK, C), lambda i: (i, 0)),
        ],
        out_specs=pl.BlockSpec((BLK, C), lambda i: (i, 0)),
        out_shape=jax.ShapeDtypeStruct((n_out, C), out_dtype),
    )(p0, p1)


def _p2_scatter(h, src3, dst3, zrows, n_sup, n_acc, rows_t, k0, k1, e_pad):
    N, C = h.shape

    @functools.partial(
        pl.kernel,
        out_type=[
            jax.ShapeDtypeStruct((n_acc, C), jnp.float32),
            jax.ShapeDtypeStruct((n_acc, C), jnp.float32),
            jax.ShapeDtypeStruct((e_pad, C), jnp.float32),
        ],
        mesh=_mesh(),
        scratch_types=[
            pltpu.VMEM((SUP, CH), jnp.int32),
            pltpu.VMEM((SUP, CH), jnp.int32),
            pltpu.VMEM((CH, C), jnp.float32),
            pltpu.VMEM((CH, C), jnp.float32),
            pltpu.VMEM_SHARED((n_acc, C), jnp.float32),
            pltpu.SemaphoreType.DMA,
            pltpu.SemaphoreType.DMA,
            pltpu.SemaphoreType.DMA,
            pltpu.SemaphoreType.DMA,
            pltpu.SemaphoreType.DMA,
            pltpu.SemaphoreType.DMA,
        ],
    )
    def k(h_hbm, src_hbm, dst_hbm, z_hbm, p0_hbm, p1_hbm, hs_hbm,
          sidx, didx, rows0, rows1, acc, semg0, semg1, semw0, semw1,
          semh0, semh1):
        rowsv = (rows0, rows1)
        semgv = (semg0, semg1)
        semwv = (semw0, semw1)
        semhv = (semh0, semh1)
        c = lax.axis_index("c")
        s = lax.axis_index("s")
        dummy = h_hbm.at[pl.ds(0, CH)]
        pltpu.sync_copy(z_hbm, acc.at[pl.ds(s * rows_t, rows_t)])
        plsc.subcore_barrier()
        n_mine = jnp.where(c == 0, k0, k1)
        start = jnp.where(c == 0, s * k0, NS * k0 + s * k1)

        def sup_body(g, _):
            gsc = start + g
            pltpu.sync_copy(src_hbm.at[gsc], sidx)
            pltpu.sync_copy(dst_hbm.at[gsc], didx)

            @pl.when(g > 0)
            def _():
                for b in range(2):
                    pltpu.make_async_copy(dummy, rowsv[b], semwv[b]).wait()
                    pltpu.make_async_copy(dummy, rowsv[b], semhv[b]).wait()

            for b in range(2):
                pltpu.async_copy(h_hbm.at[sidx.at[b]], rowsv[b], semgv[b])
            for j in range(SUP):
                b = j % 2
                if j >= 2:
                    pltpu.make_async_copy(dummy, rowsv[b], semwv[b]).wait()
                    pltpu.make_async_copy(dummy, rowsv[b], semhv[b]).wait()
                    pltpu.async_copy(h_hbm.at[sidx.at[j]], rowsv[b],
                                     semgv[b])
                pltpu.make_async_copy(dummy, rowsv[b], semgv[b]).wait()
                pltpu.async_copy(rowsv[b], acc.at[didx.at[j]], semwv[b],
                                 add=True)
                off = (gsc * SUP + j) * CH
                pltpu.async_copy(rowsv[b], hs_hbm.at[pl.ds(off, CH)],
                                 semhv[b])
            return 0

        lax.fori_loop(0, n_mine, sup_body, 0)
        for b in range(2):
            pltpu.make_async_copy(dummy, rowsv[b], semwv[b]).wait()
            pltpu.make_async_copy(dummy, rowsv[b], semhv[b]).wait()
        plsc.subcore_barrier()

        @pl.when(c == 0)
        def _():
            pltpu.sync_copy(acc.at[pl.ds(s * rows_t, rows_t)],
                            p0_hbm.at[pl.ds(s * rows_t, rows_t)])

        @pl.when(c == 1)
        def _():
            pltpu.sync_copy(acc.at[pl.ds(s * rows_t, rows_t)],
                            p1_hbm.at[pl.ds(s * rows_t, rows_t)])

    return k(h, src3, dst3, zrows)


def _p3_gather(emb, src3, dst3, n_sup, e_pad, rows_t):
    N, C = emb.shape
    dt = emb.dtype

    @functools.partial(
        pl.kernel,
        out_type=[
            jax.ShapeDtypeStruct((e_pad, C), dt),
            jax.ShapeDtypeStruct((e_pad, C), dt),
        ],
        mesh=_mesh(),
        scratch_types=[
            pltpu.VMEM((SUP, CH), jnp.int32),
            pltpu.VMEM((SUP, CH), jnp.int32),
            pltpu.VMEM((CH, C), dt),
            pltpu.VMEM((CH, C), dt),
            pltpu.VMEM_SHARED((NS * rows_t, C), jnp.float32),
            pltpu.SemaphoreType.DMA,
            pltpu.SemaphoreType.DMA,
        ],
    )
    def k(e_hbm, src_hbm, dst_hbm, f1_hbm, f2_hbm,
          sidx, didx, rows1, rows2, etab, semg, semg2):
        c = lax.axis_index("c")
        s = lax.axis_index("s")
        pltpu.sync_copy(e_hbm.at[pl.ds(s * rows_t, rows_t)],
                        etab.at[pl.ds(s * rows_t, rows_t)])
        plsc.subcore_barrier()

        def sup_body(g, _):
            gsc = (c * NS + s) * n_sup + g
            pltpu.sync_copy(src_hbm.at[gsc], sidx)
            pltpu.sync_copy(dst_hbm.at[gsc], didx)

            def chunk(j, _):
                off = (gsc * SUP + j) * CH
                g1 = pltpu.async_copy(etab.at[sidx.at[j]], rows1, semg)
                g2 = pltpu.async_copy(etab.at[didx.at[j]], rows2, semg2)
                g1.wait()
                pltpu.sync_copy(rows1, f1_hbm.at[pl.ds(off, CH)])
                g2.wait()
                pltpu.sync_copy(rows2, f2_hbm.at[pl.ds(off, CH)])
                return 0

            lax.fori_loop(0, SUP, chunk, 0)
            return 0

        lax.fori_loop(0, n_sup, sup_body, 0)

    return k(emb, src3, dst3)


def _p4_mlp(f1, f2, Wp, b1r, w2r, b2r):
    e_pad, C = f1.shape
    H = Wp.shape[2]
    BLK = 4096

    def body(f1_ref, f2_ref, w_ref, b1_ref, w2_ref, b2_ref, out_ref):
        f1b = f1_ref[...].astype(jnp.bfloat16)
        f2b = f2_ref[...].astype(jnp.bfloat16)
        dn = (((0,), (1,)), ((), ()))
        acc = lax.dot_general(w_ref[0], f1b, dn,
                              preferred_element_type=jnp.float32)
        acc += lax.dot_general(w_ref[1], f2b, dn,
                               preferred_element_type=jnp.float32)
        hbT = jnp.maximum(acc + b1_ref[...], 0.0)
        ewT = lax.dot_general(w2_ref[...], hbT, (((1,), (0,)), ((), ())),
                              preferred_element_type=jnp.float32)
        ew = jnp.maximum(ewT + b2_ref[0], 0.0)
        out_ref[...] = jnp.reshape(ew, (1, 1, BLK))

    return pl.pallas_call(
        body,
        grid=(e_pad // BLK,),
        in_specs=[
            pl.BlockSpec((BLK, C), lambda i: (i, 0)),
            pl.BlockSpec((BLK, C), lambda i: (i, 0)),
            pl.BlockSpec((2, C, H), lambda i: (0, 0, 0)),
            pl.BlockSpec((H, 1), lambda i: (0, 0)),
            pl.BlockSpec((1, H), lambda i: (0, 0)),
            pl.BlockSpec(memory_space=pltpu.MemorySpace.SMEM),
        ],
        out_specs=pl.BlockSpec((1, 1, BLK), lambda i: (i, 0, 0)),
        out_shape=jax.ShapeDtypeStruct((e_pad // BLK, 1, BLK), jnp.float32),
    )(f1, f2, Wp, b1r, w2r, b2r)


def _p5_weighted_scatter(hs, dst3, ew3, zrows, n_sup, n_acc, rows_t,
                         k0, k1):
    e_pad, C = hs.shape

    @functools.partial(
        pl.kernel,
        out_type=[
            jax.ShapeDtypeStruct((n_acc, C), jnp.float32),
            jax.ShapeDtypeStruct((n_acc, C), jnp.float32),
        ],
        mesh=_mesh(),
        scratch_types=[
            pltpu.VMEM((SUP, CH), jnp.int32),
            pltpu.VMEM((SUP, CH), jnp.float32),
            pltpu.VMEM((CH, C), jnp.float32),
            pltpu.VMEM((CH, C), jnp.float32),
            pltpu.VMEM_SHARED((n_acc, C), jnp.float32),
            pltpu.SemaphoreType.DMA,
            pltpu.SemaphoreType.DMA,
            pltpu.SemaphoreType.DMA,
            pltpu.SemaphoreType.DMA,
        ],
        compiler_params=pltpu.CompilerParams(needs_layout_passes=False),
    )
    def k(hs_hbm, dst_hbm, ew_hbm, z_hbm, p0_hbm, p1_hbm,
          didx, ewb, rows0, rows1, acc,
          semg0, semg1, semw0, semw1):
        rowsv = (rows0, rows1)
        semgv = (semg0, semg1)
        semwv = (semw0, semw1)
        c = lax.axis_index("c")
        s = lax.axis_index("s")
        dummy = hs_hbm.at[pl.ds(0, CH)]
        pltpu.sync_copy(z_hbm, acc.at[pl.ds(s * rows_t, rows_t)])
        plsc.subcore_barrier()
        n_mine = jnp.where(c == 0, k0, k1)
        start = jnp.where(c == 0, s * k0, NS * k0 + s * k1)

        def sup_body(g, _):
            gsc = start + g
            pltpu.sync_copy(dst_hbm.at[gsc], didx)
            pltpu.sync_copy(ew_hbm.at[gsc], ewb)

            @pl.when(g > 0)
            def _():
                for b in range(2):
                    pltpu.make_async_copy(dummy, rowsv[b], semwv[b]).wait()

            for b in range(2):
                pltpu.async_copy(hs_hbm.at[pl.ds((gsc * SUP + b) * CH, CH)],
                                 rowsv[b], semgv[b])
            for j in range(SUP):
                b = j % 2
                rows = rowsv[b]
                if j >= 2:
                    pltpu.make_async_copy(dummy, rows, semwv[b]).wait()
                    pltpu.async_copy(
                        hs_hbm.at[pl.ds((gsc * SUP + j) * CH, CH)], rows,
                        semgv[b])
                pltpu.make_async_copy(dummy, rows, semgv[b]).wait()
                jv = jnp.full((LANES,), j, jnp.int32)

                @plsc.parallel_loop(0, CH, unroll=4)
                def row(r):
                    sv = plsc.load_gather(
                        ewb, [jv, jnp.full((LANES,), r, jnp.int32)])
                    for q in range(C // LANES):
                        sl = pl.ds(q * LANES, LANES)
                        rows[r, sl] = rows[r, sl] * sv

                pltpu.async_copy(rows, acc.at[didx.at[j]], semwv[b],
                                 add=True)
            return 0

        lax.fori_loop(0, n_mine, sup_body, 0)
        for b in range(2):
            pltpu.make_async_copy(dummy, rowsv[b], semwv[b]).wait()
        plsc.subcore_barrier()

        @pl.when(c == 0)
        def _():
            pltpu.sync_copy(acc.at[pl.ds(s * rows_t, rows_t)],
                            p0_hbm.at[pl.ds(s * rows_t, rows_t)])

        @pl.when(c == 1)
        def _():
            pltpu.sync_copy(acc.at[pl.ds(s * rows_t, rows_t)],
                            p1_hbm.at[pl.ds(s * rows_t, rows_t)])

    return k(hs, dst3, ew3, zrows)


def kernel(x, edge_index, W_base, b_base, W1, b1, W2, b2):
    N, C = x.shape
    E = edge_index.shape[1]
    H = W1.shape[1]

    n_sup = -(-E // (NC * NS * CH * SUP))
    n_chunks = n_sup * SUP
    e_pad = NC * NS * n_chunks * CH
    rows_t = 8 * (-(-(N + 1) // (NS * 8)))
    n_acc = NS * rows_t

    src = edge_index[0].astype(jnp.int32)
    dst = edge_index[1].astype(jnp.int32)
    pad = e_pad - E
    srcp = jnp.concatenate([src, jnp.zeros((pad,), jnp.int32)])
    dstp = jnp.concatenate([dst, jnp.full((pad,), N, jnp.int32)])
    src3 = srcp.reshape(-1, SUP, CH)
    dst3 = dstp.reshape(-1, SUP, CH)
    zrows = jnp.zeros((rows_t, C), jnp.float32)

    total_sup = NC * n_sup
    k0 = (7 * total_sup) // 10
    k1 = total_sup - k0

    h = _p1_linear(x, W_base, b_base.reshape(1, C))
    ep0, ep1, hs = _p2_scatter(h, src3, dst3, zrows, n_sup, n_acc, rows_t,
                               k0, k1, e_pad)
    emb = _padd(ep0, ep1, n_acc)
    f1, f2 = _p3_gather(emb, src3, dst3, n_sup, e_pad, rows_t)

    Wp = jnp.stack([W1[:C], W1[C:]]).astype(jnp.bfloat16)
    ew_mat = _p4_mlp(f1, f2, Wp, b1.reshape(H, 1), W2.reshape(1, H),
                     b2.reshape(1))
    ew3 = ew_mat.reshape(-1, SUP, CH)

    ksym = total_sup // 2
    lp0, lp1 = _p5_weighted_scatter(hs, dst3, ew3, zrows,
                                    n_sup, n_acc, rows_t, ksym,
                                    total_sup - ksym)
    return _padd(lp0, lp1, N)
